# baseline (speedup 1.0000x reference)
import jax
import jax.numpy as jnp
from jax.experimental import pallas as pl

_NUM_EXPERTS = 8
_BT = 2048
_STREAMS = 2


def _top2(logits):
    lt = logits.T
    sub = jax.lax.broadcasted_iota(jnp.int32, lt.shape, 0)
    m1 = jnp.max(lt, axis=0, keepdims=True)
    i1 = jnp.min(jnp.where(lt == m1, sub, _NUM_EXPERTS), axis=0, keepdims=True)
    masked = jnp.where(sub == i1, -jnp.inf, lt)
    m2 = jnp.max(masked, axis=0, keepdims=True)
    i2 = jnp.min(jnp.where(masked == m2, sub, _NUM_EXPERTS), axis=0, keepdims=True)
    e = jnp.exp(m2 - m1)
    denom = 1.0 + e
    return jnp.concatenate([1.0 / denom, e / denom], axis=0), jnp.concatenate([i1, i2], axis=0)


def _router_block(x0_ref, x1_ref, w_ref, logits_ref, prob_ref, idx_ref):
    w = w_ref[...]
    for s, x_ref in enumerate((x0_ref, x1_ref)):
        logits = jax.lax.dot_general(
            x_ref[...], w, (((1,), (1,)), ((), ())),
            preferred_element_type=jnp.float32,
        )
        logits_ref[pl.ds(s * _BT, _BT), :] = logits
        prob, idx = _top2(logits)
        prob_ref[:, pl.ds(s * _BT, _BT)] = prob
        idx_ref[:, pl.ds(s * _BT, _BT)] = idx


def kernel(hidden_states, weight):
    n_tokens, hidden = hidden_states.shape
    n_experts = weight.shape[0]
    step = _STREAMS * _BT
    grid = (n_tokens // step,)
    logits, prob_t, idx_t = pl.pallas_call(
        _router_block,
        grid=grid,
        in_specs=[
            pl.BlockSpec((_BT, hidden), lambda i: (2 * i, 0)),
            pl.BlockSpec((_BT, hidden), lambda i: (2 * i + 1, 0)),
            pl.BlockSpec((n_experts, hidden), lambda i: (0, 0)),
        ],
        out_specs=[
            pl.BlockSpec((step, n_experts), lambda i: (i, 0)),
            pl.BlockSpec((2, step), lambda i: (0, i)),
            pl.BlockSpec((2, step), lambda i: (0, i)),
        ],
        out_shape=[
            jax.ShapeDtypeStruct((n_tokens, n_experts), jnp.float32),
            jax.ShapeDtypeStruct((2, n_tokens), jnp.float32),
            jax.ShapeDtypeStruct((2, n_tokens), jnp.int32),
        ],
    )(hidden_states, hidden_states, weight)
    return (logits, prob_t.T, idx_t.T)
